# Initial kernel scaffold; baseline (speedup 1.0000x reference)
import jax
import jax.numpy as jnp
from jax.experimental import pallas as pl


def kernel(xyz, W1, b1, g1, be1, W2, b2, g2, be2, W3, b3, g3, be3):
    raise NotImplementedError("write your pallas kernel here")


# speedup vs baseline: 7.1061x; 7.1061x over previous
import functools

import jax
import jax.numpy as jnp
from jax.experimental import pallas as pl
from jax.experimental.pallas import tpu as pltpu

B = 16
N = 4096
M = 128
K = 32
P = B * M * K
EPS = 1e-5
BIG = 3.0e38


def _fps_body(xyz_ref, centers_ref):
    x0 = xyz_ref[:, 0, :]
    x1 = xyz_ref[:, 1, :]
    x2 = xyz_ref[:, 2, :]
    iota_n = jax.lax.broadcasted_iota(jnp.int32, (B, N), 1)
    iota_m = jax.lax.broadcasted_iota(jnp.int32, (B, M), 1)

    def body(i, state):
        distances, farthest = state
        sel = iota_n == farthest[:, None]
        c0 = jnp.sum(jnp.where(sel, x0, 0.0), axis=1)
        c1 = jnp.sum(jnp.where(sel, x1, 0.0), axis=1)
        c2 = jnp.sum(jnp.where(sel, x2, 0.0), axis=1)
        slot = iota_m == i
        centers_ref[:, 0, :] = jnp.where(slot, c0[:, None], centers_ref[:, 0, :])
        centers_ref[:, 1, :] = jnp.where(slot, c1[:, None], centers_ref[:, 1, :])
        centers_ref[:, 2, :] = jnp.where(slot, c2[:, None], centers_ref[:, 2, :])
        d = (x0 - c0[:, None]) ** 2 + (x1 - c1[:, None]) ** 2 + (x2 - c2[:, None]) ** 2
        distances = jnp.minimum(distances, d)
        dmax = jnp.max(distances, axis=1, keepdims=True)
        cand = jnp.where(distances == dmax, iota_n, N)
        farthest = jnp.min(cand, axis=1)
        return distances, farthest

    distances = jnp.full((B, N), BIG, dtype=jnp.float32)
    farthest = jnp.zeros((B,), dtype=jnp.int32)
    jax.lax.fori_loop(0, M, body, (distances, farthest))


def _fps(xyz_p):
    return pl.pallas_call(
        _fps_body,
        out_shape=jax.ShapeDtypeStruct((B, 3, M), jnp.float32),
    )(xyz_p)


def _knn_body(xyz_ref, cent_ref, patches_ref):
    x0 = xyz_ref[0, 0, :][None, :]
    x1 = xyz_ref[0, 1, :][None, :]
    x2 = xyz_ref[0, 2, :][None, :]
    c0 = cent_ref[0, 0, :][:, None]
    c1 = cent_ref[0, 1, :][:, None]
    c2 = cent_ref[0, 2, :][:, None]
    d = (c0 - x0) ** 2 + (c1 - x1) ** 2 + (c2 - x2) ** 2
    iota_n = jax.lax.broadcasted_iota(jnp.int32, (M, N), 1)
    for k in range(K):
        vmin = jnp.min(d, axis=1, keepdims=True)
        cand = jnp.where(d == vmin, iota_n, N)
        col = jnp.min(cand, axis=1, keepdims=True)
        selmask = iota_n == col
        p0 = jnp.sum(jnp.where(selmask, x0, 0.0), axis=1)
        p1 = jnp.sum(jnp.where(selmask, x1, 0.0), axis=1)
        p2 = jnp.sum(jnp.where(selmask, x2, 0.0), axis=1)
        patches_ref[0, 0, k * M:(k + 1) * M] = p0 - c0[:, 0]
        patches_ref[0, 1, k * M:(k + 1) * M] = p1 - c1[:, 0]
        patches_ref[0, 2, k * M:(k + 1) * M] = p2 - c2[:, 0]
        d = jnp.where(selmask, BIG, d)


def _knn(xyz_p, centers_p):
    return pl.pallas_call(
        _knn_body,
        grid=(B,),
        in_specs=[
            pl.BlockSpec((1, 3, N), lambda b: (b, 0, 0)),
            pl.BlockSpec((1, 3, M), lambda b: (b, 0, 0)),
        ],
        out_specs=pl.BlockSpec((1, 3, K * M), lambda b: (b, 0, 0)),
        out_shape=jax.ShapeDtypeStruct((B, 3, K * M), jnp.float32),
    )(xyz_p, centers_p)


def _accum_stats(y, s_ref, ss_ref, first):
    C = y.shape[0]
    s = jnp.zeros((C, M), jnp.float32)
    ss = jnp.zeros((C, M), jnp.float32)
    for k in range(K):
        blk = y[:, k * M:(k + 1) * M]
        s = s + blk
        ss = ss + blk * blk

    @pl.when(first)
    def _():
        s_ref[...] = jnp.zeros_like(s_ref)
        ss_ref[...] = jnp.zeros_like(ss_ref)

    s_ref[...] += s
    ss_ref[...] += ss


def _norm(y, s, ss, g, be):
    mean = jnp.sum(s, axis=1, keepdims=True) * (1.0 / P)
    var = jnp.sum(ss, axis=1, keepdims=True) * (1.0 / P) - mean * mean
    return g * (y - mean) * jax.lax.rsqrt(var + EPS) + be


def _mlp1_body(x_ref, w1_ref, b1_ref, y_ref, s_ref, ss_ref):
    b = pl.program_id(0)
    x = x_ref[0]
    w = w1_ref[...]
    y = b1_ref[...] + w[:, 0:1] * x[0:1, :] + w[:, 1:2] * x[1:2, :] + w[:, 2:3] * x[2:3, :]
    y_ref[0] = y
    _accum_stats(y, s_ref, ss_ref, b == 0)


def _mlp2_body(y1_ref, s1_ref, ss1_ref, w2_ref, b2_ref, g1_ref, be1_ref,
               y_ref, s_ref, ss_ref):
    b = pl.program_id(0)
    x = jax.nn.relu(_norm(y1_ref[0], s1_ref[...], ss1_ref[...], g1_ref[...], be1_ref[...]))
    y = jnp.dot(w2_ref[...], x, preferred_element_type=jnp.float32) + b2_ref[...]
    y_ref[0] = y
    _accum_stats(y, s_ref, ss_ref, b == 0)


def _mlp3_body(y2_ref, s2_ref, ss2_ref, w3_ref, b3_ref, g2_ref, be2_ref,
               t_ref, s_ref, ss_ref):
    b = pl.program_id(0)
    x = jax.nn.relu(_norm(y2_ref[0], s2_ref[...], ss2_ref[...], g2_ref[...], be2_ref[...]))
    y = jnp.dot(w3_ref[...], x, preferred_element_type=jnp.float32) + b3_ref[...]
    _accum_stats(y, s_ref, ss_ref, b == 0)
    t = y[:, 0:M]
    for k in range(1, K):
        t = jnp.maximum(t, y[:, k * M:(k + 1) * M])
    t_ref[0] = t


def _finish_body(t_ref, s3_ref, ss3_ref, g3_ref, be3_ref, out_ref):
    s = s3_ref[...]
    ss = ss3_ref[...]
    mean = jnp.sum(s, axis=1, keepdims=True) * (1.0 / P)
    var = jnp.sum(ss, axis=1, keepdims=True) * (1.0 / P) - mean * mean
    scale = g3_ref[...] * jax.lax.rsqrt(var + EPS)
    off = be3_ref[...] - mean * scale
    for b in range(B):
        tok = jax.nn.relu(t_ref[b] * scale + off)
        out_ref[b] = tok.T


def _stats_spec(c):
    return pl.BlockSpec((c, M), lambda b: (0, 0))


def _full_spec(shape):
    nd = len(shape)
    return pl.BlockSpec(shape, lambda b: (0,) * nd)


def _mlp(x_p, W1, b1, g1, be1, W2, b2, g2, be2, W3, b3, g3, be3):
    col = lambda v: v[:, None]
    y1, s1, ss1 = pl.pallas_call(
        _mlp1_body,
        grid=(B,),
        in_specs=[
            pl.BlockSpec((1, 3, K * M), lambda b: (b, 0, 0)),
            _full_spec((64, 3)),
            _full_spec((64, 1)),
        ],
        out_specs=[
            pl.BlockSpec((1, 64, K * M), lambda b: (b, 0, 0)),
            _stats_spec(64),
            _stats_spec(64),
        ],
        out_shape=[
            jax.ShapeDtypeStruct((B, 64, K * M), jnp.float32),
            jax.ShapeDtypeStruct((64, M), jnp.float32),
            jax.ShapeDtypeStruct((64, M), jnp.float32),
        ],
    )(x_p, W1.T, col(b1))

    y2, s2, ss2 = pl.pallas_call(
        _mlp2_body,
        grid=(B,),
        in_specs=[
            pl.BlockSpec((1, 64, K * M), lambda b: (b, 0, 0)),
            _stats_spec(64), _stats_spec(64),
            _full_spec((128, 64)),
            _full_spec((128, 1)),
            _full_spec((64, 1)), _full_spec((64, 1)),
        ],
        out_specs=[
            pl.BlockSpec((1, 128, K * M), lambda b: (b, 0, 0)),
            _stats_spec(128),
            _stats_spec(128),
        ],
        out_shape=[
            jax.ShapeDtypeStruct((B, 128, K * M), jnp.float32),
            jax.ShapeDtypeStruct((128, M), jnp.float32),
            jax.ShapeDtypeStruct((128, M), jnp.float32),
        ],
    )(y1, s1, ss1, W2.T, col(b2), col(g1), col(be1))

    t, s3, ss3 = pl.pallas_call(
        _mlp3_body,
        grid=(B,),
        in_specs=[
            pl.BlockSpec((1, 128, K * M), lambda b: (b, 0, 0)),
            _stats_spec(128), _stats_spec(128),
            _full_spec((384, 128)),
            _full_spec((384, 1)),
            _full_spec((128, 1)), _full_spec((128, 1)),
        ],
        out_specs=[
            pl.BlockSpec((1, 384, M), lambda b: (b, 0, 0)),
            _stats_spec(384),
            _stats_spec(384),
        ],
        out_shape=[
            jax.ShapeDtypeStruct((B, 384, M), jnp.float32),
            jax.ShapeDtypeStruct((384, M), jnp.float32),
            jax.ShapeDtypeStruct((384, M), jnp.float32),
        ],
    )(y2, s2, ss2, W3.T, col(b3), col(g2), col(be2))

    tokens = pl.pallas_call(
        _finish_body,
        out_shape=jax.ShapeDtypeStruct((B, M, 384), jnp.float32),
    )(t, s3, ss3, col(g3), col(be3))
    return tokens


@jax.jit
def kernel(xyz, W1, b1, g1, be1, W2, b2, g2, be2, W3, b3, g3, be3):
    xyz_p = jnp.transpose(xyz, (0, 2, 1))
    centers_p = _fps(xyz_p)
    patches = _knn(xyz_p, centers_p)
    tokens = _mlp(patches, W1, b1, g1, be1, W2, b2, g2, be2, W3, b3, g3, be3)
    centers = jnp.transpose(centers_p, (0, 2, 1))
    return tokens, centers


# speedup vs baseline: 13.7275x; 1.9318x over previous
import functools

import jax
import jax.numpy as jnp
from jax import lax
from jax.experimental import pallas as pl
from jax.experimental.pallas import tpu as pltpu
from jax.experimental.pallas import tpu_sc as plsc

B = 16
N = 4096
M = 128
K = 32
P = B * M * K
EPS = 1e-5
BIG = 3.0e38

NC = 2
NS = 16
CHUNK = (K * M) // 2


def _fps_body(xyz_ref, centers_ref):
    x0 = xyz_ref[:, 0, :]
    x1 = xyz_ref[:, 1, :]
    x2 = xyz_ref[:, 2, :]
    iota_n = jax.lax.broadcasted_iota(jnp.int32, (B, N), 1)
    iota_m = jax.lax.broadcasted_iota(jnp.int32, (B, M), 1)

    def body(i, state):
        distances, farthest = state
        sel = iota_n == farthest[:, None]
        c0 = jnp.sum(jnp.where(sel, x0, 0.0), axis=1)
        c1 = jnp.sum(jnp.where(sel, x1, 0.0), axis=1)
        c2 = jnp.sum(jnp.where(sel, x2, 0.0), axis=1)
        slot = iota_m == i
        centers_ref[:, 0, :] = jnp.where(slot, c0[:, None], centers_ref[:, 0, :])
        centers_ref[:, 1, :] = jnp.where(slot, c1[:, None], centers_ref[:, 1, :])
        centers_ref[:, 2, :] = jnp.where(slot, c2[:, None], centers_ref[:, 2, :])
        d = (x0 - c0[:, None]) ** 2 + (x1 - c1[:, None]) ** 2 + (x2 - c2[:, None]) ** 2
        distances = jnp.minimum(distances, d)
        dmax = jnp.max(distances, axis=1, keepdims=True)
        cand = jnp.where(distances == dmax, iota_n, N)
        farthest = jnp.min(cand, axis=1)
        return distances, farthest

    distances = jnp.full((B, N), BIG, dtype=jnp.float32)
    farthest = jnp.zeros((B,), dtype=jnp.int32)
    jax.lax.fori_loop(0, M, body, (distances, farthest))


def _fps(xyz_p):
    return pl.pallas_call(
        _fps_body,
        out_shape=jax.ShapeDtypeStruct((B, 3, M), jnp.float32),
    )(xyz_p)


def _knn_body(xyz_ref, cent_ref, idx_ref):
    x0 = xyz_ref[0, 0, :][None, :]
    x1 = xyz_ref[0, 1, :][None, :]
    x2 = xyz_ref[0, 2, :][None, :]
    c0 = cent_ref[0, 0, :][:, None]
    c1 = cent_ref[0, 1, :][:, None]
    c2 = cent_ref[0, 2, :][:, None]
    d = (c0 - x0) ** 2 + (c1 - x1) ** 2 + (c2 - x2) ** 2
    iota_n = jax.lax.broadcasted_iota(jnp.int32, (M, N), 1)
    for k in range(K):
        vmin = jnp.min(d, axis=1, keepdims=True)
        cand = jnp.where(d == vmin, iota_n, N)
        col = jnp.min(cand, axis=1, keepdims=True)
        idx_ref[0, k, :] = col[:, 0]
        d = jnp.where(iota_n == col, BIG, d)


def _knn_idx(xyz_p, centers_p):
    return pl.pallas_call(
        _knn_body,
        grid=(B,),
        in_specs=[
            pl.BlockSpec((1, 3, N), lambda b: (b, 0, 0)),
            pl.BlockSpec((1, 3, M), lambda b: (b, 0, 0)),
        ],
        out_specs=pl.BlockSpec((1, K, M), lambda b: (b, 0, 0)),
        out_shape=jax.ShapeDtypeStruct((B, K, M), jnp.int32),
    )(xyz_p, centers_p)


def _sc_gather_body(xyz_hbm, cent_hbm, idx_hbm, out_hbm,
                    x0_v, x1_v, x2_v, c0_v, c1_v, c2_v, i_v,
                    o0_v, o1_v, o2_v):
    wid = lax.axis_index("s") * NC + lax.axis_index("c")
    b = wid // 2
    off = (wid % 2) * CHUNK
    xs = (x0_v, x1_v, x2_v)
    cs = (c0_v, c1_v, c2_v)
    os = (o0_v, o1_v, o2_v)
    for c in range(3):
        pltpu.sync_copy(xyz_hbm.at[pl.ds(pl.multiple_of((b * 3 + c) * N, N), N)], xs[c])
        pltpu.sync_copy(cent_hbm.at[pl.ds(pl.multiple_of((b * 3 + c) * M, M), M)], cs[c])
    pltpu.sync_copy(idx_hbm.at[pl.ds(pl.multiple_of(b * (K * M) + off, CHUNK), CHUNK)], i_v)

    for i in range(CHUNK // 16):
        vidx = i_v[pl.ds(i * 16, 16)]
        mslot = (i % 8) * 16
        for c in range(3):
            g = plsc.load_gather(xs[c], [vidx])
            ce = cs[c][pl.ds(mslot, 16)]
            os[c][pl.ds(i * 16, 16)] = g - ce
    for c in range(3):
        pltpu.sync_copy(
            os[c],
            out_hbm.at[pl.ds(pl.multiple_of((b * 3 + c) * (K * M) + off, CHUNK), CHUNK)],
        )


def _sc_gather(xyz_p, centers_p, idx_flat):
    kfn = pl.kernel(
        _sc_gather_body,
        out_type=jax.ShapeDtypeStruct((B * 3 * K * M,), jnp.float32),
        mesh=plsc.VectorSubcoreMesh(
            core_axis_name="c", subcore_axis_name="s", num_cores=NC, num_subcores=NS
        ),
        compiler_params=pltpu.CompilerParams(needs_layout_passes=False),
        scratch_types=[
            pltpu.VMEM((N,), jnp.float32), pltpu.VMEM((N,), jnp.float32),
            pltpu.VMEM((N,), jnp.float32),
            pltpu.VMEM((M,), jnp.float32), pltpu.VMEM((M,), jnp.float32),
            pltpu.VMEM((M,), jnp.float32),
            pltpu.VMEM((CHUNK,), jnp.int32),
            pltpu.VMEM((CHUNK,), jnp.float32), pltpu.VMEM((CHUNK,), jnp.float32),
            pltpu.VMEM((CHUNK,), jnp.float32),
        ],
    )
    out = kfn(xyz_p.reshape(B * 3 * N), centers_p.reshape(B * 3 * M), idx_flat)
    return out.reshape(B, 3, K * M)


def _accum_stats(y, s_ref, ss_ref, first):
    C = y.shape[0]
    s = jnp.zeros((C, M), jnp.float32)
    ss = jnp.zeros((C, M), jnp.float32)
    for k in range(K):
        blk = y[:, k * M:(k + 1) * M]
        s = s + blk
        ss = ss + blk * blk

    @pl.when(first)
    def _():
        s_ref[...] = jnp.zeros_like(s_ref)
        ss_ref[...] = jnp.zeros_like(ss_ref)

    s_ref[...] += s
    ss_ref[...] += ss


def _norm(y, s, ss, g, be):
    mean = jnp.sum(s, axis=1, keepdims=True) * (1.0 / P)
    var = jnp.sum(ss, axis=1, keepdims=True) * (1.0 / P) - mean * mean
    return g * (y - mean) * jax.lax.rsqrt(var + EPS) + be


def _mlp1_body(x_ref, w1_ref, b1_ref, y_ref, s_ref, ss_ref):
    b = pl.program_id(0)
    x = x_ref[0]
    w = w1_ref[...]
    y = b1_ref[...] + w[:, 0:1] * x[0:1, :] + w[:, 1:2] * x[1:2, :] + w[:, 2:3] * x[2:3, :]
    y_ref[0] = y
    _accum_stats(y, s_ref, ss_ref, b == 0)


def _mlp2_body(y1_ref, s1_ref, ss1_ref, w2_ref, b2_ref, g1_ref, be1_ref,
               y_ref, s_ref, ss_ref):
    b = pl.program_id(0)
    x = jax.nn.relu(_norm(y1_ref[0], s1_ref[...], ss1_ref[...], g1_ref[...], be1_ref[...]))
    y = jnp.dot(w2_ref[...], x, preferred_element_type=jnp.float32) + b2_ref[...]
    y_ref[0] = y
    _accum_stats(y, s_ref, ss_ref, b == 0)


def _mlp3_body(y2_ref, s2_ref, ss2_ref, w3_ref, b3_ref, g2_ref, be2_ref,
               t_ref, s_ref, ss_ref):
    b = pl.program_id(0)
    x = jax.nn.relu(_norm(y2_ref[0], s2_ref[...], ss2_ref[...], g2_ref[...], be2_ref[...]))
    y = jnp.dot(w3_ref[...], x, preferred_element_type=jnp.float32) + b3_ref[...]
    _accum_stats(y, s_ref, ss_ref, b == 0)
    t = y[:, 0:M]
    for k in range(1, K):
        t = jnp.maximum(t, y[:, k * M:(k + 1) * M])
    t_ref[0] = t


def _finish_body(t_ref, s3_ref, ss3_ref, g3_ref, be3_ref, out_ref):
    s = s3_ref[...]
    ss = ss3_ref[...]
    mean = jnp.sum(s, axis=1, keepdims=True) * (1.0 / P)
    var = jnp.sum(ss, axis=1, keepdims=True) * (1.0 / P) - mean * mean
    scale = g3_ref[...] * jax.lax.rsqrt(var + EPS)
    off = be3_ref[...] - mean * scale
    for b in range(B):
        tok = jax.nn.relu(t_ref[b] * scale + off)
        out_ref[b] = tok.T


def _stats_spec(c):
    return pl.BlockSpec((c, M), lambda b: (0, 0))


def _full_spec(shape):
    nd = len(shape)
    return pl.BlockSpec(shape, lambda b: (0,) * nd)


def _mlp(x_p, W1, b1, g1, be1, W2, b2, g2, be2, W3, b3, g3, be3):
    col = lambda v: v[:, None]
    y1, s1, ss1 = pl.pallas_call(
        _mlp1_body,
        grid=(B,),
        in_specs=[
            pl.BlockSpec((1, 3, K * M), lambda b: (b, 0, 0)),
            _full_spec((64, 3)),
            _full_spec((64, 1)),
        ],
        out_specs=[
            pl.BlockSpec((1, 64, K * M), lambda b: (b, 0, 0)),
            _stats_spec(64),
            _stats_spec(64),
        ],
        out_shape=[
            jax.ShapeDtypeStruct((B, 64, K * M), jnp.float32),
            jax.ShapeDtypeStruct((64, M), jnp.float32),
            jax.ShapeDtypeStruct((64, M), jnp.float32),
        ],
    )(x_p, W1.T, col(b1))

    y2, s2, ss2 = pl.pallas_call(
        _mlp2_body,
        grid=(B,),
        in_specs=[
            pl.BlockSpec((1, 64, K * M), lambda b: (b, 0, 0)),
            _stats_spec(64), _stats_spec(64),
            _full_spec((128, 64)),
            _full_spec((128, 1)),
            _full_spec((64, 1)), _full_spec((64, 1)),
        ],
        out_specs=[
            pl.BlockSpec((1, 128, K * M), lambda b: (b, 0, 0)),
            _stats_spec(128),
            _stats_spec(128),
        ],
        out_shape=[
            jax.ShapeDtypeStruct((B, 128, K * M), jnp.float32),
            jax.ShapeDtypeStruct((128, M), jnp.float32),
            jax.ShapeDtypeStruct((128, M), jnp.float32),
        ],
    )(y1, s1, ss1, W2.T, col(b2), col(g1), col(be1))

    t, s3, ss3 = pl.pallas_call(
        _mlp3_body,
        grid=(B,),
        in_specs=[
            pl.BlockSpec((1, 128, K * M), lambda b: (b, 0, 0)),
            _stats_spec(128), _stats_spec(128),
            _full_spec((384, 128)),
            _full_spec((384, 1)),
            _full_spec((128, 1)), _full_spec((128, 1)),
        ],
        out_specs=[
            pl.BlockSpec((1, 384, M), lambda b: (b, 0, 0)),
            _stats_spec(384),
            _stats_spec(384),
        ],
        out_shape=[
            jax.ShapeDtypeStruct((B, 384, M), jnp.float32),
            jax.ShapeDtypeStruct((384, M), jnp.float32),
            jax.ShapeDtypeStruct((384, M), jnp.float32),
        ],
    )(y2, s2, ss2, W3.T, col(b3), col(g2), col(be2))

    tokens = pl.pallas_call(
        _finish_body,
        out_shape=jax.ShapeDtypeStruct((B, M, 384), jnp.float32),
    )(t, s3, ss3, col(g3), col(be3))
    return tokens


@jax.jit
def kernel(xyz, W1, b1, g1, be1, W2, b2, g2, be2, W3, b3, g3, be3):
    xyz_p = jnp.transpose(xyz, (0, 2, 1))
    centers_p = _fps(xyz_p)
    idx = _knn_idx(xyz_p, centers_p)
    patches = _sc_gather(xyz_p, centers_p, idx.reshape(B * K * M))
    tokens = _mlp(patches, W1, b1, g1, be1, W2, b2, g2, be2, W3, b3, g3, be3)
    centers = jnp.transpose(centers_p, (0, 2, 1))
    return tokens, centers


# speedup vs baseline: 17.7755x; 1.2949x over previous
import jax
import jax.numpy as jnp
from jax import lax
from jax.experimental import pallas as pl
from jax.experimental.pallas import tpu as pltpu
from jax.experimental.pallas import tpu_sc as plsc

B = 16
N = 4096
M = 128
K = 32
P = B * M * K
EPS = 1e-5
BIG = 3.0e38

NC = 2
NS = 16
CHUNK = (K * M) // 2


def _fpsknn_body(xyz_ref, centers_ref, idx_ref):
    i = pl.program_id(0)

    @pl.when(i == 0)
    def _fps():
        x0 = xyz_ref[:, 0, :]
        x1 = xyz_ref[:, 1, :]
        x2 = xyz_ref[:, 2, :]
        iota_n = jax.lax.broadcasted_iota(jnp.int32, (B, N), 1).astype(jnp.float32)
        iota_m = jax.lax.broadcasted_iota(jnp.int32, (B, M), 1)

        def body(t, state):
            distances, farthest = state
            sel = iota_n == farthest[:, None]
            c0 = jnp.sum(jnp.where(sel, x0, 0.0), axis=1)
            c1 = jnp.sum(jnp.where(sel, x1, 0.0), axis=1)
            c2 = jnp.sum(jnp.where(sel, x2, 0.0), axis=1)
            slot = iota_m == t
            centers_ref[:, 0, :] = jnp.where(slot, c0[:, None], centers_ref[:, 0, :])
            centers_ref[:, 1, :] = jnp.where(slot, c1[:, None], centers_ref[:, 1, :])
            centers_ref[:, 2, :] = jnp.where(slot, c2[:, None], centers_ref[:, 2, :])
            d = (x0 - c0[:, None]) ** 2 + (x1 - c1[:, None]) ** 2 + (x2 - c2[:, None]) ** 2
            distances = jnp.minimum(distances, d)
            dmax = jnp.max(distances, axis=1, keepdims=True)
            cand = jnp.where(distances == dmax, iota_n, float(N))
            farthest = jnp.min(cand, axis=1)
            return distances, farthest

        distances = jnp.full((B, N), BIG, dtype=jnp.float32)
        farthest = jnp.zeros((B,), dtype=jnp.float32)
        jax.lax.fori_loop(0, M, body, (distances, farthest))

    @pl.when(i > 0)
    def _knn():
        b = i - 1
        x0 = xyz_ref[b, 0, :][None, :]
        x1 = xyz_ref[b, 1, :][None, :]
        x2 = xyz_ref[b, 2, :][None, :]
        c0 = centers_ref[b, 0, :][:, None]
        c1 = centers_ref[b, 1, :][:, None]
        c2 = centers_ref[b, 2, :][:, None]
        d = (c0 - x0) ** 2 + (c1 - x1) ** 2 + (c2 - x2) ** 2
        iota_n = jax.lax.broadcasted_iota(jnp.int32, (M, N), 1).astype(jnp.float32)
        for k in range(K):
            vmin = jnp.min(d, axis=1, keepdims=True)
            cand = jnp.where(d == vmin, iota_n, float(N))
            col = jnp.min(cand, axis=1, keepdims=True)
            idx_ref[0, k, :] = col[:, 0].astype(jnp.int32)
            d = jnp.where(iota_n == col, BIG, d)


def _fpsknn(xyz_p):
    return pl.pallas_call(
        _fpsknn_body,
        grid=(B + 1,),
        in_specs=[pl.BlockSpec((B, 3, N), lambda i: (0, 0, 0))],
        out_specs=[
            pl.BlockSpec((B, 3, M), lambda i: (0, 0, 0)),
            pl.BlockSpec((1, K, M), lambda i: (jnp.maximum(i - 1, 0), 0, 0)),
        ],
        out_shape=[
            jax.ShapeDtypeStruct((B, 3, M), jnp.float32),
            jax.ShapeDtypeStruct((B, K, M), jnp.int32),
        ],
    )(xyz_p)


def _sc_gather_body(xyz_hbm, cent_hbm, idx_hbm, out_hbm,
                    x0_v, x1_v, x2_v, c0_v, c1_v, c2_v, i_v,
                    o0_v, o1_v, o2_v):
    wid = lax.axis_index("s") * NC + lax.axis_index("c")
    b = wid // 2
    off = (wid % 2) * CHUNK
    xs = (x0_v, x1_v, x2_v)
    cs = (c0_v, c1_v, c2_v)
    os = (o0_v, o1_v, o2_v)
    for c in range(3):
        pltpu.sync_copy(xyz_hbm.at[pl.ds(pl.multiple_of((b * 3 + c) * N, N), N)], xs[c])
        pltpu.sync_copy(cent_hbm.at[pl.ds(pl.multiple_of((b * 3 + c) * M, M), M)], cs[c])
    pltpu.sync_copy(idx_hbm.at[pl.ds(pl.multiple_of(b * (K * M) + off, CHUNK), CHUNK)], i_v)

    for i in range(CHUNK // 16):
        vidx = i_v[pl.ds(i * 16, 16)]
        mslot = (i % 8) * 16
        for c in range(3):
            g = plsc.load_gather(xs[c], [vidx])
            ce = cs[c][pl.ds(mslot, 16)]
            os[c][pl.ds(i * 16, 16)] = g - ce

    for c in range(3):
        pltpu.sync_copy(
            os[c],
            out_hbm.at[pl.ds(pl.multiple_of((b * 3 + c) * (K * M) + off, CHUNK), CHUNK)],
        )


def _sc_gather(xyz_p, centers_p, idx_flat):
    kfn = pl.kernel(
        _sc_gather_body,
        out_type=jax.ShapeDtypeStruct((B * 3 * K * M,), jnp.float32),
        mesh=plsc.VectorSubcoreMesh(
            core_axis_name="c", subcore_axis_name="s", num_cores=NC, num_subcores=NS
        ),
        compiler_params=pltpu.CompilerParams(needs_layout_passes=False),
        scratch_types=[
            pltpu.VMEM((N,), jnp.float32), pltpu.VMEM((N,), jnp.float32),
            pltpu.VMEM((N,), jnp.float32),
            pltpu.VMEM((M,), jnp.float32), pltpu.VMEM((M,), jnp.float32),
            pltpu.VMEM((M,), jnp.float32),
            pltpu.VMEM((CHUNK,), jnp.int32),
            pltpu.VMEM((CHUNK,), jnp.float32), pltpu.VMEM((CHUNK,), jnp.float32),
            pltpu.VMEM((CHUNK,), jnp.float32),
        ],
    )
    out = kfn(xyz_p.reshape(B * 3 * N), centers_p.reshape(B * 3 * M), idx_flat)
    return out.reshape(B, 3, K * M)


def _part_stats(y):
    C = y.shape[0]
    s = jnp.zeros((C, M), jnp.float32)
    ss = jnp.zeros((C, M), jnp.float32)
    for k in range(K):
        blk = y[:, k * M:(k + 1) * M]
        s = s + blk
        ss = ss + blk * blk
    return s, ss


def _norm2(y, s, ss, g, be):
    mean = jnp.sum(s, axis=1, keepdims=True) * (1.0 / P)
    var = jnp.sum(ss, axis=1, keepdims=True) * (1.0 / P) - mean * mean
    return g * (y - mean) * jax.lax.rsqrt(var + EPS) + be


def _layer1(x, w, b1):
    return b1 + w[:, 0:1] * x[0:1, :] + w[:, 1:2] * x[1:2, :] + w[:, 2:3] * x[2:3, :]


def _fused_mlp_body(x_ref, w1_ref, b1_ref, g1_ref, be1_ref,
                    w2_ref, b2_ref, g2_ref, be2_ref,
                    w3_ref, b3_ref, g3_ref, be3_ref,
                    out_ref,
                    y2_s, t_s, s1, ss1, s2, ss2, s3, ss3):
    i = pl.program_id(0)

    @pl.when(i == 0)
    def _():
        s1[...] = jnp.zeros_like(s1)
        ss1[...] = jnp.zeros_like(ss1)
        s2[...] = jnp.zeros_like(s2)
        ss2[...] = jnp.zeros_like(ss2)
        s3[...] = jnp.zeros_like(s3)
        ss3[...] = jnp.zeros_like(ss3)

    @pl.when(i < B)
    def _():
        y1 = _layer1(x_ref[0], w1_ref[...], b1_ref[...])
        ps, pss = _part_stats(y1)
        s1[...] += ps
        ss1[...] += pss

    @pl.when(jnp.logical_and(i >= B, i < 2 * B))
    def _():
        b = i - B
        y1 = _layer1(x_ref[0], w1_ref[...], b1_ref[...])
        x2 = jax.nn.relu(_norm2(y1, s1[...], ss1[...], g1_ref[...], be1_ref[...]))
        y2 = jnp.dot(w2_ref[...], x2, preferred_element_type=jnp.float32) + b2_ref[...]
        y2_s[pl.ds(b, 1)] = y2[None]
        ps, pss = _part_stats(y2)
        s2[...] += ps
        ss2[...] += pss

    @pl.when(jnp.logical_and(i >= 2 * B, i < 3 * B))
    def _():
        b = i - 2 * B
        y2 = y2_s[pl.ds(b, 1)][0]
        x3 = jax.nn.relu(_norm2(y2, s2[...], ss2[...], g2_ref[...], be2_ref[...]))
        y3 = jnp.dot(w3_ref[...], x3, preferred_element_type=jnp.float32) + b3_ref[...]
        ps, pss = _part_stats(y3)
        s3[...] += ps
        ss3[...] += pss
        t = y3[:, 0:M]
        for k in range(1, K):
            t = jnp.maximum(t, y3[:, k * M:(k + 1) * M])
        t_s[pl.ds(b, 1)] = t[None]

    @pl.when(i == 3 * B)
    def _():
        mean = jnp.sum(s3[...], axis=1, keepdims=True) * (1.0 / P)
        var = jnp.sum(ss3[...], axis=1, keepdims=True) * (1.0 / P) - mean * mean
        scale = g3_ref[...] * jax.lax.rsqrt(var + EPS)
        off = be3_ref[...] - mean * scale
        for b in range(B):
            tok = jax.nn.relu(t_s[b] * scale + off)
            out_ref[b] = tok.T


def _x_im(i):
    b = jnp.where(i < B, i, jnp.where(i < 2 * B, i - B, jnp.where(i < 3 * B, i - 2 * B, 0)))
    return (b, 0, 0)


def _fullc(shape):
    nd = len(shape)
    return pl.BlockSpec(shape, lambda i: (0,) * nd)


def _mlp_fused(x_p, W1, b1, g1, be1, W2, b2, g2, be2, W3, b3, g3, be3):
    col = lambda v: v[:, None]
    return pl.pallas_call(
        _fused_mlp_body,
        grid=(3 * B + 1,),
        in_specs=[
            pl.BlockSpec((1, 3, K * M), _x_im),
            _fullc((64, 3)), _fullc((64, 1)), _fullc((64, 1)), _fullc((64, 1)),
            _fullc((128, 64)), _fullc((128, 1)), _fullc((128, 1)), _fullc((128, 1)),
            _fullc((384, 128)), _fullc((384, 1)), _fullc((384, 1)), _fullc((384, 1)),
        ],
        out_specs=pl.BlockSpec((B, M, 384), lambda i: (0, 0, 0)),
        out_shape=jax.ShapeDtypeStruct((B, M, 384), jnp.float32),
        scratch_shapes=[
            pltpu.VMEM((B, 128, K * M), jnp.float32),
            pltpu.VMEM((B, 384, M), jnp.float32),
            pltpu.VMEM((64, M), jnp.float32), pltpu.VMEM((64, M), jnp.float32),
            pltpu.VMEM((128, M), jnp.float32), pltpu.VMEM((128, M), jnp.float32),
            pltpu.VMEM((384, M), jnp.float32), pltpu.VMEM((384, M), jnp.float32),
        ],
    )(x_p, W1.T, col(b1), col(g1), col(be1),
      W2.T, col(b2), col(g2), col(be2),
      W3.T, col(b3), col(g3), col(be3))


@jax.jit
def kernel(xyz, W1, b1, g1, be1, W2, b2, g2, be2, W3, b3, g3, be3):
    xyz_p = jnp.transpose(xyz, (0, 2, 1))
    centers_p, idx = _fpsknn(xyz_p)
    patches = _sc_gather(xyz_p, centers_p, idx.reshape(B * K * M))
    tokens = _mlp_fused(patches, W1, b1, g1, be1, W2, b2, g2, be2, W3, b3, g3, be3)
    centers = jnp.transpose(centers_p, (0, 2, 1))
    return tokens, centers


# speedup vs baseline: 17.9331x; 1.0089x over previous
import jax
import jax.numpy as jnp
from jax import lax
from jax.experimental import pallas as pl
from jax.experimental.pallas import tpu as pltpu
from jax.experimental.pallas import tpu_sc as plsc

B = 16
N = 4096
M = 128
K = 32
P = B * M * K
EPS = 1e-5
BIG = 3.0e38

NC = 2
NS = 16
CHUNK = (K * M) // 2


def _fpsknn_body(xyz_ref, centers_ref, idx_ref):
    i = pl.program_id(0)

    @pl.when(i == 0)
    def _fps():
        x0 = xyz_ref[:, 0, :]
        x1 = xyz_ref[:, 1, :]
        x2 = xyz_ref[:, 2, :]
        iota_n = jax.lax.broadcasted_iota(jnp.int32, (B, N), 1).astype(jnp.float32)
        iota_m = jax.lax.broadcasted_iota(jnp.int32, (B, M), 1)

        def body(t, state):
            distances, farthest = state
            sel = iota_n == farthest[:, None]
            c0 = jnp.sum(jnp.where(sel, x0, 0.0), axis=1)
            c1 = jnp.sum(jnp.where(sel, x1, 0.0), axis=1)
            c2 = jnp.sum(jnp.where(sel, x2, 0.0), axis=1)
            slot = iota_m == t
            centers_ref[:, 0, :] = jnp.where(slot, c0[:, None], centers_ref[:, 0, :])
            centers_ref[:, 1, :] = jnp.where(slot, c1[:, None], centers_ref[:, 1, :])
            centers_ref[:, 2, :] = jnp.where(slot, c2[:, None], centers_ref[:, 2, :])
            d = (x0 - c0[:, None]) ** 2 + (x1 - c1[:, None]) ** 2 + (x2 - c2[:, None]) ** 2
            distances = jnp.minimum(distances, d)
            dmax = jnp.max(distances, axis=1, keepdims=True)
            cand = jnp.where(distances == dmax, iota_n, float(N))
            farthest = jnp.min(cand, axis=1)
            return distances, farthest

        distances = jnp.full((B, N), BIG, dtype=jnp.float32)
        farthest = jnp.zeros((B,), dtype=jnp.float32)
        jax.lax.fori_loop(0, M, body, (distances, farthest))

    @pl.when(i > 0)
    def _knn():
        b = i - 1
        x0 = xyz_ref[b, 0, :][None, :]
        x1 = xyz_ref[b, 1, :][None, :]
        x2 = xyz_ref[b, 2, :][None, :]
        c0 = centers_ref[b, 0, :][:, None]
        c1 = centers_ref[b, 1, :][:, None]
        c2 = centers_ref[b, 2, :][:, None]
        d = (c0 - x0) ** 2 + (c1 - x1) ** 2 + (c2 - x2) ** 2
        iota_n = jax.lax.broadcasted_iota(jnp.int32, (M, N), 1).astype(jnp.float32)
        for k in range(K):
            vmin = jnp.min(d, axis=1, keepdims=True)
            cand = jnp.where(d == vmin, iota_n, float(N))
            col = jnp.min(cand, axis=1, keepdims=True)
            idx_ref[0, k, :] = col[:, 0].astype(jnp.int32)
            d = jnp.where(iota_n == col, BIG, d)


def _fpsknn(xyz_p):
    return pl.pallas_call(
        _fpsknn_body,
        grid=(B + 1,),
        in_specs=[pl.BlockSpec((B, 3, N), lambda i: (0, 0, 0))],
        out_specs=[
            pl.BlockSpec((B, 3, M), lambda i: (0, 0, 0)),
            pl.BlockSpec((1, K, M), lambda i: (jnp.maximum(i - 1, 0), 0, 0)),
        ],
        out_shape=[
            jax.ShapeDtypeStruct((B, 3, M), jnp.float32),
            jax.ShapeDtypeStruct((B, K, M), jnp.int32),
        ],
    )(xyz_p)


def _sc_gather_body(xyz_hbm, cent_hbm, idx_hbm, out_hbm,
                    x0_v, x1_v, x2_v, c0_v, c1_v, c2_v, i_v,
                    o0_v, o1_v, o2_v):
    wid = lax.axis_index("s") * NC + lax.axis_index("c")
    b = wid // 2
    off = (wid % 2) * CHUNK
    xs = (x0_v, x1_v, x2_v)
    cs = (c0_v, c1_v, c2_v)
    os = (o0_v, o1_v, o2_v)
    for c in range(3):
        pltpu.sync_copy(xyz_hbm.at[pl.ds(pl.multiple_of((b * 3 + c) * N, N), N)], xs[c])
        pltpu.sync_copy(cent_hbm.at[pl.ds(pl.multiple_of((b * 3 + c) * M, M), M)], cs[c])
    pltpu.sync_copy(idx_hbm.at[pl.ds(pl.multiple_of(b * (K * M) + off, CHUNK), CHUNK)], i_v)

    for i in range(CHUNK // 16):
        vidx = i_v[pl.ds(i * 16, 16)]
        mslot = (i % 8) * 16
        for c in range(3):
            g = plsc.load_gather(xs[c], [vidx])
            ce = cs[c][pl.ds(mslot, 16)]
            os[c][pl.ds(i * 16, 16)] = g - ce

    for c in range(3):
        pltpu.sync_copy(
            os[c],
            out_hbm.at[pl.ds(pl.multiple_of((b * 3 + c) * (K * M) + off, CHUNK), CHUNK)],
        )


def _sc_gather(xyz_p, centers_p, idx_flat):
    kfn = pl.kernel(
        _sc_gather_body,
        out_type=jax.ShapeDtypeStruct((B * 3 * K * M,), jnp.float32),
        mesh=plsc.VectorSubcoreMesh(
            core_axis_name="c", subcore_axis_name="s", num_cores=NC, num_subcores=NS
        ),
        compiler_params=pltpu.CompilerParams(needs_layout_passes=False),
        scratch_types=[
            pltpu.VMEM((N,), jnp.float32), pltpu.VMEM((N,), jnp.float32),
            pltpu.VMEM((N,), jnp.float32),
            pltpu.VMEM((M,), jnp.float32), pltpu.VMEM((M,), jnp.float32),
            pltpu.VMEM((M,), jnp.float32),
            pltpu.VMEM((CHUNK,), jnp.int32),
            pltpu.VMEM((CHUNK,), jnp.float32), pltpu.VMEM((CHUNK,), jnp.float32),
            pltpu.VMEM((CHUNK,), jnp.float32),
        ],
    )
    out = kfn(xyz_p.reshape(B * 3 * N), centers_p.reshape(B * 3 * M), idx_flat)
    return out.reshape(B, 3, K * M)


def _part_stats(y, nk):
    C = y.shape[0]
    s = jnp.zeros((C, M), jnp.float32)
    ss = jnp.zeros((C, M), jnp.float32)
    for k in range(nk):
        blk = y[:, k * M:(k + 1) * M]
        s = s + blk
        ss = ss + blk * blk
    return s, ss


def _norm2(y, s, ss, g, be):
    mean = jnp.sum(s, axis=1, keepdims=True) * (1.0 / P)
    var = jnp.sum(ss, axis=1, keepdims=True) * (1.0 / P) - mean * mean
    return g * (y - mean) * jax.lax.rsqrt(var + EPS) + be


def _layer1(x, w, b1):
    return b1 + w[:, 0:1] * x[0:1, :] + w[:, 1:2] * x[1:2, :] + w[:, 2:3] * x[2:3, :]


NB = 2
G = B // NB
KM = K * M


def _fused_mlp_body(x_ref, w1_ref, b1_ref, g1_ref, be1_ref,
                    w2_ref, b2_ref, g2_ref, be2_ref,
                    w3_ref, b3_ref, g3_ref, be3_ref,
                    out_ref,
                    y2_s, t_s, s1, ss1, s2, ss2, s3, ss3):
    i = pl.program_id(0)

    @pl.when(i == 0)
    def _():
        s1[...] = jnp.zeros_like(s1)
        ss1[...] = jnp.zeros_like(ss1)
        s2[...] = jnp.zeros_like(s2)
        ss2[...] = jnp.zeros_like(ss2)
        s3[...] = jnp.zeros_like(s3)
        ss3[...] = jnp.zeros_like(ss3)

    @pl.when(i < G)
    def _():
        x = jnp.concatenate([x_ref[0], x_ref[1]], axis=1)
        y1 = _layer1(x, w1_ref[...], b1_ref[...])
        ps, pss = _part_stats(y1, NB * K)
        s1[...] += ps
        ss1[...] += pss

    @pl.when(jnp.logical_and(i >= G, i < 2 * G))
    def _():
        j = i - G
        x = jnp.concatenate([x_ref[0], x_ref[1]], axis=1)
        y1 = _layer1(x, w1_ref[...], b1_ref[...])
        x2 = jax.nn.relu(_norm2(y1, s1[...], ss1[...], g1_ref[...], be1_ref[...]))
        y2 = jnp.dot(w2_ref[...], x2, preferred_element_type=jnp.float32) + b2_ref[...]
        y2_s[pl.ds(NB * j, 1)] = y2[:, 0:KM][None]
        y2_s[pl.ds(NB * j + 1, 1)] = y2[:, KM:2 * KM][None]
        ps, pss = _part_stats(y2, NB * K)
        s2[...] += ps
        ss2[...] += pss

    @pl.when(jnp.logical_and(i >= 2 * G, i < 3 * G))
    def _():
        j = i - 2 * G
        for h in range(NB):
            y2 = y2_s[pl.ds(NB * j + h, 1)][0]
            x3 = jax.nn.relu(_norm2(y2, s2[...], ss2[...], g2_ref[...], be2_ref[...]))
            y3 = jnp.dot(w3_ref[...], x3, preferred_element_type=jnp.float32) + b3_ref[...]
            ps, pss = _part_stats(y3, K)
            s3[...] += ps
            ss3[...] += pss
            t = y3[:, 0:M]
            for k in range(1, K):
                t = jnp.maximum(t, y3[:, k * M:(k + 1) * M])
            t_s[pl.ds(NB * j + h, 1)] = t[None]

    @pl.when(i == 3 * G)
    def _():
        mean = jnp.sum(s3[...], axis=1, keepdims=True) * (1.0 / P)
        var = jnp.sum(ss3[...], axis=1, keepdims=True) * (1.0 / P) - mean * mean
        scale = g3_ref[...] * jax.lax.rsqrt(var + EPS)
        off = be3_ref[...] - mean * scale
        for b in range(B):
            tok = jax.nn.relu(t_s[b] * scale + off)
            out_ref[b] = tok.T


def _x_im(i):
    j = jnp.where(i < G, i, jnp.where(i < 2 * G, i - G, jnp.where(i < 3 * G, i - 2 * G, 0)))
    return (j, 0, 0)


def _fullc(shape):
    nd = len(shape)
    return pl.BlockSpec(shape, lambda i: (0,) * nd)


def _mlp_fused(x_p, W1, b1, g1, be1, W2, b2, g2, be2, W3, b3, g3, be3):
    col = lambda v: v[:, None]
    return pl.pallas_call(
        _fused_mlp_body,
        grid=(3 * G + 1,),
        in_specs=[
            pl.BlockSpec((NB, 3, K * M), _x_im),
            _fullc((64, 3)), _fullc((64, 1)), _fullc((64, 1)), _fullc((64, 1)),
            _fullc((128, 64)), _fullc((128, 1)), _fullc((128, 1)), _fullc((128, 1)),
            _fullc((384, 128)), _fullc((384, 1)), _fullc((384, 1)), _fullc((384, 1)),
        ],
        out_specs=pl.BlockSpec((B, M, 384), lambda i: (0, 0, 0)),
        out_shape=jax.ShapeDtypeStruct((B, M, 384), jnp.float32),
        scratch_shapes=[
            pltpu.VMEM((B, 128, K * M), jnp.float32),
            pltpu.VMEM((B, 384, M), jnp.float32),
            pltpu.VMEM((64, M), jnp.float32), pltpu.VMEM((64, M), jnp.float32),
            pltpu.VMEM((128, M), jnp.float32), pltpu.VMEM((128, M), jnp.float32),
            pltpu.VMEM((384, M), jnp.float32), pltpu.VMEM((384, M), jnp.float32),
        ],
    )(x_p, W1.T, col(b1), col(g1), col(be1),
      W2.T, col(b2), col(g2), col(be2),
      W3.T, col(b3), col(g3), col(be3))


@jax.jit
def kernel(xyz, W1, b1, g1, be1, W2, b2, g2, be2, W3, b3, g3, be3):
    xyz_p = jnp.transpose(xyz, (0, 2, 1))
    centers_p, idx = _fpsknn(xyz_p)
    patches = _sc_gather(xyz_p, centers_p, idx.reshape(B * K * M))
    tokens = _mlp_fused(patches, W1, b1, g1, be1, W2, b2, g2, be2, W3, b3, g3, be3)
    centers = jnp.transpose(centers_p, (0, 2, 1))
    return tokens, centers
